# baseline (speedup 1.0000x reference)
import jax
import jax.numpy as jnp
from jax import lax
from jax.experimental import pallas as pl
from jax.experimental.pallas import tpu as pltpu
from jax.experimental.pallas import tpu_sc as plsc

LEVELS = 1024
TIMESTAMPS = 2048
DIM = 10000
SIGNAL_MIN = -3.0
SIGNAL_MAX = 3.0

NC = 2
NS = 16
NU = 39
DMAIN = NU * 256
U0 = 20
U1 = NU - U0
NWIDE0, NWIDE1 = 4, 3
G = 16
NGRP = TIMESTAMPS // G
SIN1 = 0.8414709848078965


def _value_to_index(value, low, high, num):
    idx = jnp.round((value - low) / (high - low) * (num - 1))
    return jnp.clip(idx, 0, num - 1).astype(jnp.int32)


def _sc_body(idx_hbm, bx, by, bz, bt, out0_hbm, out1_hbm, idx_v, idx_u, bufs,
             acc_v, out_v, sems):
    cid = lax.axis_index("c")
    s = lax.axis_index("s")
    tables = (bx, by, bz, bt)

    pltpu.sync_copy(idx_hbm, idx_v)

    def run(unit0, nu, col0, out_hbm):
        width = nu * 256

        def pf(ci, carry):
            sl16 = pl.ds(ci * 16, 16)
            for t in range(4):
                v = idx_v[t, sl16] * NU
                for u in range(nu):
                    idx_u[t, u, sl16] = v + (unit0 + u)
            return carry

        lax.fori_loop(0, TIMESTAMPS // 16, pf, 0)

        def zf(c, carry):
            acc_v[pl.ds(c * 16, 16)] = jnp.zeros((16,), jnp.float32)
            return carry

        lax.fori_loop(0, width // 16, zf, 0)

        def issue(g, slot):
            off = pl.multiple_of(g * G, 8)
            for t in range(4):
                for u in range(nu):
                    pltpu.async_copy(
                        tables[t].at[idx_u.at[t, u, pl.ds(off, G)]],
                        bufs.at[slot, t, u],
                        sems.at[slot, t, u],
                    )

        def drain(g, slot):
            off = pl.multiple_of(g * G, 8)
            for t in range(4):
                for u in range(nu):
                    pltpu.make_async_copy(
                        tables[t].at[idx_u.at[t, u, pl.ds(off, G)]],
                        bufs.at[slot, t, u],
                        sems.at[slot, t, u],
                    ).wait()

        issue(0, 0)

        def gloop(gg, carry):
            for b in range(2):
                g = gg * 2 + b

                @pl.when(g + 1 < NGRP)
                def _prefetch():
                    issue(g + 1, 1 - b)

                drain(g, b)

                for u in range(nu):

                    def cf(c, carry2, u=u):
                        base = u * 256 + c * 32
                        a0 = acc_v[pl.ds(base, 16)]
                        a1 = acc_v[pl.ds(base + 16, 16)]
                        q = pl.ds(c * 16, 16)
                        hmask = jnp.int32(-65536)
                        neg1 = jnp.float32(-1.0)
                        pos1 = jnp.float32(1.0)
                        for j in range(G):
                            sx = (bufs[b, 0, u, j, q] ^ bufs[b, 1, u, j, q]
                                  ^ bufs[b, 2, u, j, q] ^ bufs[b, 3, u, j, q])
                            lob = (sx & jnp.int32(0x8000)) != 0
                            hib = (sx & hmask) < 0
                            a0 = a0 + jnp.where(lob, neg1, pos1)
                            a1 = a1 + jnp.where(hib, neg1, pos1)
                        acc_v[pl.ds(base, 16)] = a0
                        acc_v[pl.ds(base + 16, 16)] = a1
                        return carry2

                    lax.fori_loop(0, 8, cf, 0)
            return carry

        lax.fori_loop(0, NGRP // 2, gloop, 0)

        def ff(c, carry):
            sl16 = pl.ds(c * 16, 16)
            out_v[sl16] = jnp.sign(acc_v[sl16]) * jnp.float32(SIN1)
            return carry

        lax.fori_loop(0, width // 16, ff, 0)
        pltpu.sync_copy(out_v.at[pl.ds(0, width)],
                        out_hbm.at[pl.ds(col0, width)])

    @pl.when((cid == 0) & (s < NWIDE0))
    def _c0w():
        run(s * 2, 2, pl.multiple_of(s * 512, 8), out0_hbm)

    @pl.when((cid == 0) & (s >= NWIDE0))
    def _c0n():
        run(s + NWIDE0, 1, pl.multiple_of((s + NWIDE0) * 256, 8), out0_hbm)

    @pl.when((cid == 1) & (s < NWIDE1))
    def _c1w():
        run(U0 + s * 2, 2, pl.multiple_of(s * 512, 8), out1_hbm)

    @pl.when((cid == 1) & (s >= NWIDE1))
    def _c1n():
        run(U0 + s + NWIDE1, 1, pl.multiple_of((s + NWIDE1) * 256, 8),
            out1_hbm)


def _tail_body(xi, yi, zi, ti, tx, ty, tz, tt, o_ref):
    rows_l = lax.broadcasted_iota(jnp.int32, (1, LEVELS), 1)
    rows_t = lax.broadcasted_iota(jnp.int32, (1, TIMESTAMPS), 1)

    def emb(idx_ref, tab_ref, rows):
        oh = (idx_ref[...].reshape(TIMESTAMPS, 1) == rows).astype(jnp.float32)
        return jnp.dot(oh, tab_ref[...], preferred_element_type=jnp.float32)

    ex = emb(xi, tx, rows_l)
    ey = emb(yi, ty, rows_l)
    ez = emb(zi, tz, rows_l)
    et = emb(ti, tt, rows_t)
    total = jnp.sum(ex * ey * ez * et, axis=0)
    o_ref[...] = jnp.sin(jnp.sign(total))[None]


@jax.jit
def _sc_encode(idx, bx, by, bz, bt, tailx, taily, tailz, tailt):
    mesh = plsc.VectorSubcoreMesh(
        core_axis_name="c", subcore_axis_name="s", num_cores=NC, num_subcores=NS
    )
    main = pl.kernel(
        _sc_body,
        out_type=(
            jax.ShapeDtypeStruct((U0 * 256,), jnp.float32),
            jax.ShapeDtypeStruct((U1 * 256,), jnp.float32),
        ),
        mesh=mesh,
        scratch_types=[
            pltpu.VMEM((4, TIMESTAMPS), jnp.int32),
            pltpu.VMEM((4, 2, TIMESTAMPS), jnp.int32),
            pltpu.VMEM((2, 4, 2, G, 128), jnp.int32),
            pltpu.VMEM((512,), jnp.float32),
            pltpu.VMEM((512,), jnp.float32),
            pltpu.SemaphoreType.DMA((2, 4, 2)),
        ],
    )(idx, bx, by, bz, bt)

    tail = pl.pallas_call(
        _tail_body,
        grid=(1,),
        in_specs=[
            pl.BlockSpec((TIMESTAMPS,), lambda i: (0,)),
            pl.BlockSpec((TIMESTAMPS,), lambda i: (0,)),
            pl.BlockSpec((TIMESTAMPS,), lambda i: (0,)),
            pl.BlockSpec((TIMESTAMPS,), lambda i: (0,)),
            pl.BlockSpec((LEVELS, 128), lambda i: (0, 0)),
            pl.BlockSpec((LEVELS, 128), lambda i: (0, 0)),
            pl.BlockSpec((LEVELS, 128), lambda i: (0, 0)),
            pl.BlockSpec((TIMESTAMPS, 128), lambda i: (0, 0)),
        ],
        out_specs=pl.BlockSpec((1, 128), lambda i: (0, 0)),
        out_shape=jax.ShapeDtypeStruct((1, 128), jnp.float32),
    )(idx[0], idx[1], idx[2], idx[3], tailx, taily, tailz, tailt)

    def fix(part):
        return part.reshape(-1, 2, 16).swapaxes(1, 2).reshape(-1)

    return jnp.concatenate([fix(main[0]), fix(main[1]),
                            tail[0, : DIM - DMAIN]])


def kernel(input, table_x, table_y, table_z, table_t):
    x = jnp.clip(input[:, 1], SIGNAL_MIN, SIGNAL_MAX)
    y = jnp.clip(input[:, 2], SIGNAL_MIN, SIGNAL_MAX)
    z = jnp.clip(input[:, 3], SIGNAL_MIN, SIGNAL_MAX)
    xi = _value_to_index(x, SIGNAL_MIN, SIGNAL_MAX, LEVELS)
    yi = _value_to_index(y, SIGNAL_MIN, SIGNAL_MAX, LEVELS)
    zi = _value_to_index(z, SIGNAL_MIN, SIGNAL_MAX, LEVELS)
    ti = _value_to_index(input[:, 0], 0.0, float(TIMESTAMPS), TIMESTAMPS)
    idx = jnp.stack([xi, yi, zi, ti], axis=0)

    def prep(tab):
        rows = tab.shape[0]
        m = tab[:, :DMAIN].astype(jnp.bfloat16).reshape(rows * NU, 128, 2)
        main = lax.bitcast_convert_type(m, jnp.int32)
        tail = jnp.pad(tab[:, DMAIN:], ((0, 0), (0, 128 - (DIM - DMAIN))))
        return main, tail

    bx, tailx = prep(table_x)
    by, taily = prep(table_y)
    bz, tailz = prep(table_z)
    bt, tailt = prep(table_t)
    return _sc_encode(idx, bx, by, bz, bt, tailx, taily, tailz, tailt)


# speedup vs baseline: 62.3717x; 62.3717x over previous
import jax
import jax.numpy as jnp
from jax import lax
from jax.experimental import pallas as pl
from jax.experimental.pallas import tpu as pltpu
from jax.experimental.pallas import tpu_sc as plsc

LEVELS = 1024
TIMESTAMPS = 2048
DIM = 10000
SIGNAL_MIN = -3.0
SIGNAL_MAX = 3.0

NC = 2
NS = 16
LANES = 16
DMAIN = 9984
DCORE = DMAIN // NC
NWIDE = 7
WWIDE = 384
WNARR = 256
G = 16
NGRP = TIMESTAMPS // G
SIN1 = 0.8414709848078965


def _value_to_index(value, low, high, num):
    idx = jnp.round((value - low) / (high - low) * (num - 1))
    return jnp.clip(idx, 0, num - 1).astype(jnp.int32)


def _sc_body(idx_hbm, tx, ty, tz, tt, out0_hbm, out1_hbm, idx_v, bufs, acc_v,
             sems):
    cid = lax.axis_index("c")
    s = lax.axis_index("s")
    tables = (tx, ty, tz, tt)

    pltpu.sync_copy(idx_hbm, idx_v)

    def run(col0, width, out_hbm, cbase):
        nch = width // LANES

        def zf(c, carry):
            acc_v[pl.ds(c * LANES, LANES)] = jnp.zeros((LANES,), jnp.float32)
            return carry

        lax.fori_loop(0, nch, zf, 0)

        def issue(g, slot):
            off = pl.multiple_of(g * G, 8)
            for t in range(4):
                pltpu.async_copy(
                    tables[t].at[idx_v.at[t, pl.ds(off, G)],
                                 pl.ds(pl.multiple_of(cbase + col0, 128), width)],
                    bufs.at[slot, t, :, pl.ds(0, width)],
                    sems.at[slot, t],
                )

        def drain(g, slot):
            off = pl.multiple_of(g * G, 8)
            for t in range(4):
                pltpu.make_async_copy(
                    tables[t].at[idx_v.at[t, pl.ds(off, G)],
                                 pl.ds(pl.multiple_of(cbase + col0, 128), width)],
                    bufs.at[slot, t, :, pl.ds(0, width)],
                    sems.at[slot, t],
                ).wait()

        issue(0, 0)

        def gloop(gg, carry):
            for b in range(2):
                g = gg * 2 + b

                @pl.when(g + 1 < NGRP)
                def _prefetch():
                    issue(g + 1, 1 - b)

                drain(g, b)

                def cf(c, carry2):
                    sl = pl.ds(c * LANES, LANES)
                    a = acc_v[sl]
                    for j in range(G):
                        p = bufs[b, 0, j, sl] * bufs[b, 1, j, sl]
                        p = p * bufs[b, 2, j, sl]
                        p = p * bufs[b, 3, j, sl]
                        a = a + p
                    acc_v[sl] = a
                    return carry2

                lax.fori_loop(0, nch, cf, 0)
            return carry

        lax.fori_loop(0, NGRP // 2, gloop, 0)

        def ff(c, carry):
            sl = pl.ds(c * LANES, LANES)
            acc_v[sl] = jnp.sign(acc_v[sl]) * jnp.float32(SIN1)
            return carry

        lax.fori_loop(0, nch, ff, 0)
        pltpu.sync_copy(acc_v.at[pl.ds(0, width)],
                        out_hbm.at[pl.ds(col0, width)])

    col_wide = pl.multiple_of(s * WWIDE, 128)
    col_narr = pl.multiple_of(NWIDE * (WWIDE - WNARR) + s * WNARR, 128)

    @pl.when((cid == 0) & (s < NWIDE))
    def _c0w():
        run(col_wide, WWIDE, out0_hbm, 0)

    @pl.when((cid == 0) & (s >= NWIDE))
    def _c0n():
        run(col_narr, WNARR, out0_hbm, 0)

    @pl.when((cid == 1) & (s < NWIDE))
    def _c1w():
        run(col_wide, WWIDE, out1_hbm, DCORE)

    @pl.when((cid == 1) & (s >= NWIDE))
    def _c1n():
        run(col_narr, WNARR, out1_hbm, DCORE)


def _tail_body(xi, yi, zi, ti, tx, ty, tz, tt, o_ref):
    rows_l = lax.broadcasted_iota(jnp.int32, (1, LEVELS), 1)
    rows_t = lax.broadcasted_iota(jnp.int32, (1, TIMESTAMPS), 1)

    def emb(idx_ref, tab_ref, rows):
        oh = (idx_ref[...].reshape(TIMESTAMPS, 1) == rows).astype(jnp.float32)
        return jnp.dot(oh, tab_ref[...], preferred_element_type=jnp.float32)

    ex = emb(xi, tx, rows_l)
    ey = emb(yi, ty, rows_l)
    ez = emb(zi, tz, rows_l)
    et = emb(ti, tt, rows_t)
    total = jnp.sum(ex * ey * ez * et, axis=0)
    o_ref[...] = jnp.sin(jnp.sign(total))[None]


@jax.jit
def _sc_encode(idx, table_x, table_y, table_z, table_t):
    mesh = plsc.VectorSubcoreMesh(
        core_axis_name="c", subcore_axis_name="s", num_cores=NC, num_subcores=NS
    )
    main = pl.kernel(
        _sc_body,
        out_type=(
            jax.ShapeDtypeStruct((DCORE,), jnp.float32),
            jax.ShapeDtypeStruct((DCORE,), jnp.float32),
        ),
        mesh=mesh,
        scratch_types=[
            pltpu.VMEM((4, TIMESTAMPS), jnp.int32),
            pltpu.VMEM((2, 4, G, WWIDE), jnp.float32),
            pltpu.VMEM((WWIDE,), jnp.float32),
            pltpu.SemaphoreType.DMA((2, 4)),
        ],
    )(idx, table_x, table_y, table_z, table_t)

    tail = pl.pallas_call(
        _tail_body,
        grid=(1,),
        in_specs=[
            pl.BlockSpec((TIMESTAMPS,), lambda i: (0,)),
            pl.BlockSpec((TIMESTAMPS,), lambda i: (0,)),
            pl.BlockSpec((TIMESTAMPS,), lambda i: (0,)),
            pl.BlockSpec((TIMESTAMPS,), lambda i: (0,)),
            pl.BlockSpec((LEVELS, 128), lambda i: (0, DMAIN // 128)),
            pl.BlockSpec((LEVELS, 128), lambda i: (0, DMAIN // 128)),
            pl.BlockSpec((LEVELS, 128), lambda i: (0, DMAIN // 128)),
            pl.BlockSpec((TIMESTAMPS, 128), lambda i: (0, DMAIN // 128)),
        ],
        out_specs=pl.BlockSpec((1, 128), lambda i: (0, 0)),
        out_shape=jax.ShapeDtypeStruct((1, 128), jnp.float32),
    )(idx[0], idx[1], idx[2], idx[3], table_x, table_y, table_z, table_t)

    return jnp.concatenate([main[0], main[1], tail[0, : DIM - DMAIN]])


def kernel(input, table_x, table_y, table_z, table_t):
    x = jnp.clip(input[:, 1], SIGNAL_MIN, SIGNAL_MAX)
    y = jnp.clip(input[:, 2], SIGNAL_MIN, SIGNAL_MAX)
    z = jnp.clip(input[:, 3], SIGNAL_MIN, SIGNAL_MAX)
    xi = _value_to_index(x, SIGNAL_MIN, SIGNAL_MAX, LEVELS)
    yi = _value_to_index(y, SIGNAL_MIN, SIGNAL_MAX, LEVELS)
    zi = _value_to_index(z, SIGNAL_MIN, SIGNAL_MAX, LEVELS)
    ti = _value_to_index(input[:, 0], 0.0, float(TIMESTAMPS), TIMESTAMPS)
    idx = jnp.stack([xi, yi, zi, ti], axis=0)
    return _sc_encode(idx, table_x, table_y, table_z, table_t)


# speedup vs baseline: 64.2985x; 1.0309x over previous
import jax
import jax.numpy as jnp
from jax import lax
from jax.experimental import pallas as pl
from jax.experimental.pallas import tpu as pltpu
from jax.experimental.pallas import tpu_sc as plsc

LEVELS = 1024
TIMESTAMPS = 2048
DIM = 10000
SIGNAL_MIN = -3.0
SIGNAL_MAX = 3.0

NC = 2
NS = 16
LANES = 16
DMAIN = 9984
DCORE = DMAIN // NC
NWIDE = 7
WWIDE = 384
WNARR = 256
G = 32
NGRP = TIMESTAMPS // G
SIN1 = 0.8414709848078965


def _value_to_index(value, low, high, num):
    idx = jnp.round((value - low) / (high - low) * (num - 1))
    return jnp.clip(idx, 0, num - 1).astype(jnp.int32)


def _sc_body(idx_hbm, tx, ty, tz, tt, out0_hbm, out1_hbm, idx_v, bufs, acc_v,
             sems):
    cid = lax.axis_index("c")
    s = lax.axis_index("s")
    tables = (tx, ty, tz, tt)

    pltpu.sync_copy(idx_hbm, idx_v)

    def run(col0, width, out_hbm, cbase):
        nch = width // LANES

        def zf(c, carry):
            acc_v[pl.ds(c * LANES, LANES)] = jnp.zeros((LANES,), jnp.float32)
            return carry

        lax.fori_loop(0, nch, zf, 0)

        def issue(g, slot):
            off = pl.multiple_of(g * G, 8)
            for t in range(4):
                pltpu.async_copy(
                    tables[t].at[idx_v.at[t, pl.ds(off, G)],
                                 pl.ds(pl.multiple_of(cbase + col0, 128), width)],
                    bufs.at[slot, t, :, pl.ds(0, width)],
                    sems.at[slot, t],
                )

        def drain(g, slot):
            off = pl.multiple_of(g * G, 8)
            for t in range(4):
                pltpu.make_async_copy(
                    tables[t].at[idx_v.at[t, pl.ds(off, G)],
                                 pl.ds(pl.multiple_of(cbase + col0, 128), width)],
                    bufs.at[slot, t, :, pl.ds(0, width)],
                    sems.at[slot, t],
                ).wait()

        issue(0, 0)

        def gloop(gg, carry):
            for b in range(2):
                g = gg * 2 + b

                @pl.when(g + 1 < NGRP)
                def _prefetch():
                    issue(g + 1, 1 - b)

                drain(g, b)

                def cf(c, carry2):
                    sl = pl.ds(c * LANES, LANES)
                    a = acc_v[sl]
                    for j in range(G):
                        p = bufs[b, 0, j, sl] * bufs[b, 1, j, sl]
                        p = p * bufs[b, 2, j, sl]
                        p = p * bufs[b, 3, j, sl]
                        a = a + p
                    acc_v[sl] = a
                    return carry2

                lax.fori_loop(0, nch, cf, 0)
            return carry

        lax.fori_loop(0, NGRP // 2, gloop, 0)

        def ff(c, carry):
            sl = pl.ds(c * LANES, LANES)
            acc_v[sl] = jnp.sign(acc_v[sl]) * jnp.float32(SIN1)
            return carry

        lax.fori_loop(0, nch, ff, 0)
        pltpu.sync_copy(acc_v.at[pl.ds(0, width)],
                        out_hbm.at[pl.ds(col0, width)])

    col_wide = pl.multiple_of(s * WWIDE, 128)
    col_narr = pl.multiple_of(NWIDE * (WWIDE - WNARR) + s * WNARR, 128)

    @pl.when((cid == 0) & (s < NWIDE))
    def _c0w():
        run(col_wide, WWIDE, out0_hbm, 0)

    @pl.when((cid == 0) & (s >= NWIDE))
    def _c0n():
        run(col_narr, WNARR, out0_hbm, 0)

    @pl.when((cid == 1) & (s < NWIDE))
    def _c1w():
        run(col_wide, WWIDE, out1_hbm, DCORE)

    @pl.when((cid == 1) & (s >= NWIDE))
    def _c1n():
        run(col_narr, WNARR, out1_hbm, DCORE)


def _tail_body(xi, yi, zi, ti, tx, ty, tz, tt, o_ref):
    rows_l = lax.broadcasted_iota(jnp.int32, (1, LEVELS), 1)
    rows_t = lax.broadcasted_iota(jnp.int32, (1, TIMESTAMPS), 1)

    def emb(idx_ref, tab_ref, rows):
        oh = (idx_ref[...].reshape(TIMESTAMPS, 1) == rows).astype(jnp.float32)
        return jnp.dot(oh, tab_ref[...], preferred_element_type=jnp.float32)

    ex = emb(xi, tx, rows_l)
    ey = emb(yi, ty, rows_l)
    ez = emb(zi, tz, rows_l)
    et = emb(ti, tt, rows_t)
    total = jnp.sum(ex * ey * ez * et, axis=0)
    o_ref[...] = jnp.sin(jnp.sign(total))[None]


@jax.jit
def _sc_encode(idx, table_x, table_y, table_z, table_t):
    mesh = plsc.VectorSubcoreMesh(
        core_axis_name="c", subcore_axis_name="s", num_cores=NC, num_subcores=NS
    )
    main = pl.kernel(
        _sc_body,
        out_type=(
            jax.ShapeDtypeStruct((DCORE,), jnp.float32),
            jax.ShapeDtypeStruct((DCORE,), jnp.float32),
        ),
        mesh=mesh,
        scratch_types=[
            pltpu.VMEM((4, TIMESTAMPS), jnp.int32),
            pltpu.VMEM((2, 4, G, WWIDE), jnp.float32),
            pltpu.VMEM((WWIDE,), jnp.float32),
            pltpu.SemaphoreType.DMA((2, 4)),
        ],
    )(idx, table_x, table_y, table_z, table_t)

    tail = pl.pallas_call(
        _tail_body,
        grid=(1,),
        in_specs=[
            pl.BlockSpec((TIMESTAMPS,), lambda i: (0,)),
            pl.BlockSpec((TIMESTAMPS,), lambda i: (0,)),
            pl.BlockSpec((TIMESTAMPS,), lambda i: (0,)),
            pl.BlockSpec((TIMESTAMPS,), lambda i: (0,)),
            pl.BlockSpec((LEVELS, 128), lambda i: (0, DMAIN // 128)),
            pl.BlockSpec((LEVELS, 128), lambda i: (0, DMAIN // 128)),
            pl.BlockSpec((LEVELS, 128), lambda i: (0, DMAIN // 128)),
            pl.BlockSpec((TIMESTAMPS, 128), lambda i: (0, DMAIN // 128)),
        ],
        out_specs=pl.BlockSpec((1, 128), lambda i: (0, 0)),
        out_shape=jax.ShapeDtypeStruct((1, 128), jnp.float32),
    )(idx[0], idx[1], idx[2], idx[3], table_x, table_y, table_z, table_t)

    return jnp.concatenate([main[0], main[1], tail[0, : DIM - DMAIN]])


def kernel(input, table_x, table_y, table_z, table_t):
    x = jnp.clip(input[:, 1], SIGNAL_MIN, SIGNAL_MAX)
    y = jnp.clip(input[:, 2], SIGNAL_MIN, SIGNAL_MAX)
    z = jnp.clip(input[:, 3], SIGNAL_MIN, SIGNAL_MAX)
    xi = _value_to_index(x, SIGNAL_MIN, SIGNAL_MAX, LEVELS)
    yi = _value_to_index(y, SIGNAL_MIN, SIGNAL_MAX, LEVELS)
    zi = _value_to_index(z, SIGNAL_MIN, SIGNAL_MAX, LEVELS)
    ti = _value_to_index(input[:, 0], 0.0, float(TIMESTAMPS), TIMESTAMPS)
    idx = jnp.stack([xi, yi, zi, ti], axis=0)
    return _sc_encode(idx, table_x, table_y, table_z, table_t)
